# baseline (speedup 1.0000x reference)
import functools

import jax
import jax.numpy as jnp
from jax import lax
from jax.experimental import pallas as pl
from jax.experimental.pallas import tpu as pltpu
from jax.experimental.pallas import tpu_sc as plsc

EMB = 256
NUM_LAYER = 5
B = 64
N = 10000
E = 160000

NP = 10240
TILE = 1024
NT = NP // TILE
EP = 163840
ECHUNK = 128
NCHUNKS = EP // (16 * ECHUNK)
TRASH = 10200
PADB = 127
_BNS = float(1.0 + 1e-5) ** -0.5


CAP = 86016
SCH = 2048
NSTG = EP // 2 // SCH
NBKT = 16


def _make_edge_compact():
    mesh = plsc.VectorSubcoreMesh(core_axis_name="c", subcore_axis_name="s")

    @functools.partial(
        pl.kernel,
        mesh=mesh,
        out_type=[
            jax.ShapeDtypeStruct((32, CAP), jnp.int32),
            jax.ShapeDtypeStruct((32, CAP), jnp.int32),
            jax.ShapeDtypeStruct((32, 16), jnp.int32),
        ],
        scratch_types=[
            pltpu.VMEM((SCH,), jnp.int32),
            pltpu.VMEM((SCH,), jnp.int32),
            pltpu.VMEM((SCH + 32,), jnp.int32),
            pltpu.VMEM((SCH + 32,), jnp.int32),
            pltpu.VMEM((16,), jnp.int32),
        ],
        compiler_params=pltpu.CompilerParams(needs_layout_passes=False),
    )
    def k(src_hbm, dst_hbm, csrc, cdstl, cnts, se_v, de_v, sbuf, dbuf, cv):
        cid = lax.axis_index("c")
        sid = lax.axis_index("s")
        row = cid * NBKT + sid
        lo = sid * 640

        def flush(nf):
            pltpu.sync_copy(sbuf.at[pl.ds(0, SCH)],
                            csrc.at[row, pl.ds(nf * SCH, SCH)])
            pltpu.sync_copy(dbuf.at[pl.ds(0, SCH)],
                            cdstl.at[row, pl.ds(nf * SCH, SCH)])

        def stage(st, carry):
            pltpu.sync_copy(src_hbm.at[cid, st], se_v)
            pltpu.sync_copy(dst_hbm.at[cid, st], de_v)

            def vec(i, carry):
                cnt, nf = carry
                s16 = se_v[pl.ds(i * 16, 16)]
                d16 = de_v[pl.ds(i * 16, 16)]
                dl16 = d16 - lo
                mi = (jnp.where(dl16 >= 0, 1, 0)
                      * jnp.where(dl16 < 640, 1, 0)).astype(jnp.int32)
                iot = lax.broadcasted_iota(jnp.int32, (16,), 0)
                ps = mi
                gd = lax.GatherDimensionNumbers(
                    offset_dims=(), collapsed_slice_dims=(0,),
                    start_index_map=(0,))
                for kk in (1, 2, 4, 8):
                    sh = lax.gather(
                        ps, jnp.maximum(iot - kk, 0)[:, None], gd,
                        slice_sizes=(1,),
                        mode=lax.GatherScatterMode.PROMISE_IN_BOUNDS)
                    ps = ps + sh * jnp.where(iot >= kk, 1, 0)
                pos = ps - 1
                idx = jnp.minimum(cnt + pos + (1 - mi) * 4096, SCH + 24)
                plsc.store_scatter(sbuf, [idx], s16)
                plsc.store_scatter(dbuf, [idx], dl16)
                cnt = cnt + ps[15]
                do_f = cnt >= SCH

                @pl.when(do_f)
                def _():
                    flush(nf)
                    sbuf[pl.ds(0, 16)] = sbuf[pl.ds(SCH, 16)]
                    dbuf[pl.ds(0, 16)] = dbuf[pl.ds(SCH, 16)]
                return (jnp.where(do_f, cnt - SCH, cnt),
                        jnp.where(do_f, nf + 1, nf))
            return lax.fori_loop(0, SCH // 16, vec, carry)

        cnt, nf = lax.fori_loop(0, NSTG, stage,
                                (jnp.int32(0), jnp.int32(0)))
        flush(nf)
        cv[...] = jnp.broadcast_to(nf * SCH + cnt, (16,))
        pltpu.sync_copy(cv, cnts.at[row])

    return k


@functools.cache
def _edge_compact_kernel():
    return _make_edge_compact()


def _make_edge_segsum():
    mesh = plsc.VectorSubcoreMesh(core_axis_name="c", subcore_axis_name="s")

    @functools.partial(
        pl.kernel,
        mesh=mesh,
        out_type=jax.ShapeDtypeStruct((NP, EMB), jnp.float32),
        scratch_types=[
            pltpu.VMEM((648, 128), jnp.float32),
            pltpu.VMEM((2, ECHUNK, 128), jnp.float32),
            pltpu.VMEM((2, ECHUNK), jnp.int32),
            pltpu.VMEM((2, ECHUNK), jnp.int32),
            pltpu.VMEM((16,), jnp.int32),
        ] + [pltpu.SemaphoreType.DMA] * 2,
        compiler_params=pltpu.CompilerParams(needs_layout_passes=False),
    )
    def k(ha_hbm, hb_hbm, csrc, cdstl, cnts, out_hbm,
          acc, bufs, sidx, didx, cv, sg0, sg1):
        cid = lax.axis_index("c")
        sid = lax.axis_index("s")
        gsem = (sg0, sg1)

        def zr(r, _):
            for kk in range(8):
                acc[r, pl.ds(kk * 16, 16)] = jnp.zeros((16,), jnp.float32)
            return 0
        lax.fori_loop(0, 648, zr, 0)

        def issue(half, c, b, h_hbm):
            row = half * NBKT + sid
            pltpu.sync_copy(csrc.at[row, pl.ds(c * ECHUNK, ECHUNK)],
                            sidx.at[b])
            pltpu.sync_copy(cdstl.at[row, pl.ds(c * ECHUNK, ECHUNK)],
                            didx.at[b])
            pltpu.async_copy(h_hbm.at[sidx.at[b]], bufs.at[b], gsem[b])

        def gwait(b, h_hbm):
            pltpu.make_async_copy(h_hbm.at[sidx.at[b]], bufs.at[b],
                                  gsem[b]).wait()

        def accum(b, nrem):
            def grp(ii, _):
                base = ii * 16
                dl16 = didx[b, pl.ds(base, 16)]
                for j in range(16):
                    dl = jnp.where(base + j < nrem, dl16[j], 640)
                    for kk in range(8):
                        plsc.addupdate(acc.at[dl, pl.ds(kk * 16, 16)],
                                       bufs[b, base + j, pl.ds(kk * 16, 16)])
                return 0
            lax.fori_loop(0, ECHUNK // 16, grp, 0)

        def run(h_hbm):
            for half in range(2):
                row = half * NBKT + sid
                pltpu.sync_copy(cnts.at[row], cv)
                n = cv[...][0]
                nch = (n + ECHUNK - 1) // ECHUNK

                @pl.when(nch >= 1)
                def _():
                    issue(half, 0, 0, h_hbm)

                @pl.when(nch >= 2)
                def _():
                    issue(half, 1, 1, h_hbm)

                def pairbody(p, _):
                    for b in range(2):
                        c = 2 * p + b

                        @pl.when(c < nch)
                        def _():
                            gwait(b, h_hbm)
                            nrem = jnp.minimum(ECHUNK, n - c * ECHUNK)
                            accum(b, nrem)

                            @pl.when(c + 2 < nch)
                            def _():
                                issue(half, c + 2, b, h_hbm)
                    return 0
                lax.fori_loop(0, (nch + 1) // 2, pairbody, 0)

        @pl.when(cid == 0)
        def _():
            run(ha_hbm)

        @pl.when(cid == 1)
        def _():
            run(hb_hbm)

        pltpu.sync_copy(
            acc.at[pl.ds(0, 640)],
            out_hbm.at[pl.ds(sid * 640, 640), pl.ds(cid * 128, 128)])

    return k


@functools.cache
def _edge_segsum_kernel():
    return _make_edge_segsum()


def _edge_compact(src3, dst3):
    return _edge_compact_kernel()(src3, dst3)


def _edge_segsum(ha, hb, csrc, cdstl, cnts):
    return _edge_segsum_kernel()(ha, hb, csrc, cdstl, cnts)


def _dot(a, b):
    return jnp.dot(a, b, preferred_element_type=jnp.float32)


def _oht(batch_ref):
    bt = batch_ref[0]
    cls = lax.broadcasted_iota(jnp.int32, (B, 1), 0)
    return (cls == bt).astype(jnp.float32)


def _pre_body(h_ref, vn_ref, batch_ref, ha_ref, hb_ref, seg_ref):
    oht = _oht(batch_ref)
    h = h_ref[...]
    vng = lax.dot_general(oht, vn_ref[...], (((0,), (0,)), ((), ())),
                          preferred_element_type=jnp.float32)
    hin = h + vng
    ha_ref[...] = hin[:, :128]
    hb_ref[...] = hin[:, 128:]

    @pl.when(pl.program_id(0) == 0)
    def _():
        seg_ref[...] = jnp.zeros_like(seg_ref)
    seg_ref[...] += _dot(oht, h)


def _tc_pre(h, vn, batch3):
    return pl.pallas_call(
        _pre_body,
        grid=(NT,),
        in_specs=[
            pl.BlockSpec((TILE, EMB), lambda i: (i, 0)),
            pl.BlockSpec((B, EMB), lambda i: (0, 0)),
            pl.BlockSpec((1, 1, TILE), lambda i: (i, 0, 0)),
        ],
        out_specs=[
            pl.BlockSpec((TILE, 128), lambda i: (i, 0)),
            pl.BlockSpec((TILE, 128), lambda i: (i, 0)),
            pl.BlockSpec((B, EMB), lambda i: (0, 0)),
        ],
        out_shape=[
            jax.ShapeDtypeStruct((NP, 128), jnp.float32),
            jax.ShapeDtypeStruct((NP, 128), jnp.float32),
            jax.ShapeDtypeStruct((B, EMB), jnp.float32),
        ],
    )(h, vn, batch3)


def _mlp_body(ha_ref, hb_ref, agg_ref, w1_ref, b1_ref, w2_ref, b2_ref,
              eps_ref, out_ref, *, final):
    hin = jnp.concatenate([ha_ref[...], hb_ref[...]], axis=1)
    z = eps_ref[0, 0] * hin + agg_ref[...]
    a = jnp.maximum(_dot(z, w1_ref[...]) + b1_ref[0:1, :], 0.0)
    o = _dot(a, w2_ref[...]) + b2_ref[0:1, :]
    out_ref[...] = o if final else jnp.maximum(o, 0.0)


def _tc_mlp(ha, hb, agg, w1t, b1, w2t, b2, epsv, final):
    return pl.pallas_call(
        functools.partial(_mlp_body, final=final),
        grid=(NT,),
        in_specs=[
            pl.BlockSpec((TILE, 128), lambda i: (i, 0)),
            pl.BlockSpec((TILE, 128), lambda i: (i, 0)),
            pl.BlockSpec((TILE, EMB), lambda i: (i, 0)),
            pl.BlockSpec((EMB, 2 * EMB), lambda i: (0, 0)),
            pl.BlockSpec((8, 2 * EMB), lambda i: (0, 0)),
            pl.BlockSpec((2 * EMB, EMB), lambda i: (0, 0)),
            pl.BlockSpec((8, EMB), lambda i: (0, 0)),
            pl.BlockSpec((8, 128), lambda i: (0, 0)),
        ],
        out_specs=pl.BlockSpec((TILE, EMB), lambda i: (i, 0)),
        out_shape=jax.ShapeDtypeStruct((NP, EMB), jnp.float32),
    )(ha, hb, agg, w1t, b1, w2t, b2, epsv)


def _vn_body(seg_ref, vn_ref, w1_ref, b1_ref, w2_ref, b2_ref, out_ref):
    tmp = seg_ref[...] + vn_ref[...]
    t = jnp.maximum(_dot(tmp, w1_ref[...]) + b1_ref[0:1, :], 0.0)
    out_ref[...] = jnp.maximum(_dot(t, w2_ref[...]) + b2_ref[0:1, :], 0.0)


def _tc_vn(seg, vn, w1t, b1, w2t, b2):
    return pl.pallas_call(
        _vn_body,
        out_shape=jax.ShapeDtypeStruct((B, EMB), jnp.float32),
    )(seg, vn, w1t, b1, w2t, b2)


def _p1_body(h_ref, w1_ref, cb1_ref, wp_ref, pt_ref):
    i = pl.program_id(0)
    h = h_ref[...]
    hc = jnp.maximum(_dot(h, w1_ref[...]) + cb1_ref[0:1, :], 0.0)
    pm = _dot(hc, wp_ref[...])
    sm = _dot(h, wp_ref[...])
    p = pm[:, 0:1] + cb1_ref[1, 0]
    sim = sm[:, 1:2]
    rows = lax.broadcasted_iota(jnp.int32, (TILE, 1), 0) + i * TILE
    valid = rows < N
    pcol = jnp.where(valid, p, 0.0)
    scol = jnp.where(valid, sim, -1e30)
    pt_ref[...] = jnp.concatenate(
        [pcol, scol, jnp.zeros((TILE, 126), jnp.float32)], axis=1)


def _tc_p1(h5, cw1t, cb1x, wp):
    return pl.pallas_call(
        _p1_body,
        grid=(NT,),
        in_specs=[
            pl.BlockSpec((TILE, EMB), lambda i: (i, 0)),
            pl.BlockSpec((EMB, EMB), lambda i: (0, 0)),
            pl.BlockSpec((8, EMB), lambda i: (0, 0)),
            pl.BlockSpec((EMB, 128), lambda i: (0, 0)),
        ],
        out_specs=pl.BlockSpec((TILE, 128), lambda i: (i, 0)),
        out_shape=jax.ShapeDtypeStruct((NP, 128), jnp.float32),
    )(h5, cw1t, cb1x, wp)


def _p2_body(pt_ref, out_ref):
    p = pt_ref[:, 0:1]
    sim = pt_ref[:, 1:2]
    m = jnp.max(sim)
    s = jnp.sum(jnp.exp(sim - m))
    pres = jnp.sum((p > 0.0).astype(jnp.float32)) / jnp.float32(N)
    row = lax.broadcasted_iota(jnp.int32, (8, 128), 0)
    col = lax.broadcasted_iota(jnp.int32, (8, 128), 1)
    o = jnp.where(col == 0, m, jnp.where(col == 1, s, pres))
    out_ref[...] = jnp.where(row == 0, o, 0.0)


def _tc_p2(pt):
    return pl.pallas_call(
        _p2_body,
        out_shape=jax.ShapeDtypeStruct((8, 128), jnp.float32),
    )(pt)


def _p3_body(h_ref, pt_ref, st_ref, batch_ref, static_ref, seg_ref, cnt_ref):
    oht = _oht(batch_ref)
    h = h_ref[...]
    m = st_ref[0, 0]
    s = st_ref[0, 1]
    w = jnp.exp(pt_ref[:, 1:2] - m) / s
    t = h * w
    nrm = jnp.sqrt(jnp.sum(t * t, axis=1, keepdims=True))
    static = t / jnp.maximum(nrm, 1e-12)
    static_ref[...] = static

    @pl.when(pl.program_id(0) == 0)
    def _():
        seg_ref[...] = jnp.zeros_like(seg_ref)
        cnt_ref[...] = jnp.zeros_like(cnt_ref)
    seg_ref[...] += _dot(oht, static)
    cnt_ref[...] += _dot(oht, jnp.ones((TILE, 128), jnp.float32))


def _tc_p3(h5, pt, stats, batch3):
    return pl.pallas_call(
        _p3_body,
        grid=(NT,),
        in_specs=[
            pl.BlockSpec((TILE, EMB), lambda i: (i, 0)),
            pl.BlockSpec((TILE, 128), lambda i: (i, 0)),
            pl.BlockSpec((8, 128), lambda i: (0, 0)),
            pl.BlockSpec((1, 1, TILE), lambda i: (i, 0, 0)),
        ],
        out_specs=[
            pl.BlockSpec((TILE, EMB), lambda i: (i, 0)),
            pl.BlockSpec((B, EMB), lambda i: (0, 0)),
            pl.BlockSpec((B, 128), lambda i: (0, 0)),
        ],
        out_shape=[
            jax.ShapeDtypeStruct((NP, EMB), jnp.float32),
            jax.ShapeDtypeStruct((B, EMB), jnp.float32),
            jax.ShapeDtypeStruct((B, 128), jnp.float32),
        ],
    )(h5, pt, stats, batch3)


def _p5_body(static_ref, seg_ref, cnt_ref, batch_ref, var_ref):
    oht = _oht(batch_ref)
    cnt = jnp.maximum(cnt_ref[:, 0:1], 1.0)
    seg_mean = seg_ref[...] / cnt
    nm = lax.dot_general(oht, seg_mean, (((0,), (0,)), ((), ())),
                         preferred_element_type=jnp.float32)
    diff = static_ref[...] - nm

    @pl.when(pl.program_id(0) == 0)
    def _():
        var_ref[...] = jnp.zeros_like(var_ref)
    var_ref[...] += _dot(oht, diff * diff)


def _tc_p5(static, seg, cnt, batch3):
    return pl.pallas_call(
        _p5_body,
        grid=(NT,),
        in_specs=[
            pl.BlockSpec((TILE, EMB), lambda i: (i, 0)),
            pl.BlockSpec((B, EMB), lambda i: (0, 0)),
            pl.BlockSpec((B, 128), lambda i: (0, 0)),
            pl.BlockSpec((1, 1, TILE), lambda i: (i, 0, 0)),
        ],
        out_specs=pl.BlockSpec((B, EMB), lambda i: (0, 0)),
        out_shape=jax.ShapeDtypeStruct((B, EMB), jnp.float32),
    )(static, seg, cnt, batch3)


def _p6_body(static_ref, pt_ref, u1_ref, noise_ref, seg_ref, cnt_ref, var_ref,
             batch_ref, pool_ref, kl2_ref, kl1_ref):
    oht = _oht(batch_ref)
    onc = lax.dot_general(oht, jnp.ones((B, 1), jnp.float32),
                          (((0,), (0,)), ((), ())),
                          preferred_element_type=jnp.float32)
    counts = cnt_ref[:, 0:1]
    cnt = jnp.maximum(counts, 1.0)
    seg_mean = seg_ref[...] / cnt
    var = var_ref[...] / jnp.maximum(counts - 1.0, 1.0)
    seg_std = jnp.sqrt(jnp.maximum(var, 0.0))
    dg = functools.partial(lax.dot_general,
                           dimension_numbers=(((0,), (0,)), ((), ())),
                           preferred_element_type=jnp.float32)
    node_mean = dg(oht, seg_mean)
    node_std = dg(oht, seg_std)
    static = static_ref[...]

    bias = 0.0001
    u = u1_ref[:, 0:1]
    eps_ = (bias - (1.0 - bias)) * u + (1.0 - bias)
    gate = jnp.log(eps_) - jnp.log(1.0 - eps_) + pt_ref[:, 0:1]
    lam = jax.nn.sigmoid(gate) * onc
    nm = lam * static + (1.0 - lam) * node_mean
    ns = (1.0 - lam) * node_std
    noisy = nm + noise_ref[...] * ns
    e2 = 1e-07
    kl2t = ((nm - node_mean) / (node_std + e2)) ** 2
    kl1t = jnp.mean((ns / (node_std + e2)) ** 2, axis=1, keepdims=True) * onc

    @pl.when(pl.program_id(0) == 0)
    def _():
        pool_ref[...] = jnp.zeros_like(pool_ref)
        kl2_ref[...] = jnp.zeros_like(kl2_ref)
        kl1_ref[...] = jnp.zeros_like(kl1_ref)
    pool_ref[...] += _dot(oht, noisy)
    kl2_ref[...] += _dot(oht, kl2t)
    kl1_ref[:, 0:1] += _dot(oht, kl1t)


def _tc_p6(static, pt, u1p, noise, seg, cnt, var, batch3):
    return pl.pallas_call(
        _p6_body,
        grid=(NT,),
        in_specs=[
            pl.BlockSpec((TILE, EMB), lambda i: (i, 0)),
            pl.BlockSpec((TILE, 128), lambda i: (i, 0)),
            pl.BlockSpec((TILE, 128), lambda i: (i, 0)),
            pl.BlockSpec((TILE, EMB), lambda i: (i, 0)),
            pl.BlockSpec((B, EMB), lambda i: (0, 0)),
            pl.BlockSpec((B, 128), lambda i: (0, 0)),
            pl.BlockSpec((B, EMB), lambda i: (0, 0)),
            pl.BlockSpec((1, 1, TILE), lambda i: (i, 0, 0)),
        ],
        out_specs=[
            pl.BlockSpec((B, EMB), lambda i: (0, 0)),
            pl.BlockSpec((B, EMB), lambda i: (0, 0)),
            pl.BlockSpec((B, 128), lambda i: (0, 0)),
        ],
        out_shape=[
            jax.ShapeDtypeStruct((B, EMB), jnp.float32),
            jax.ShapeDtypeStruct((B, EMB), jnp.float32),
            jax.ShapeDtypeStruct((B, 128), jnp.float32),
        ],
    )(static, pt, u1p, noise, seg, cnt, var, batch3)


def _p7_body(pool_ref, cnt_ref, kl2_ref, kl1_ref, pw_ref, pat_ref,
             pooled_ref, sc_ref):
    cnt = jnp.maximum(cnt_ref[:, 0:1], 1.0)
    pooled = pool_ref[...] / cnt
    pooled_ref[...] = pooled
    kl_loss = (jnp.sum(kl2_ref[...])
               + EMB * 0.5 * jnp.sum(kl1_ref[:, 0:1])) / (B * EMB)
    pred = _dot(pooled, pw_ref[...])[:, 0:1] + pat_ref[1, 0]
    dmat = pat_ref[0:1, :] - pred
    ppl = jnp.mean(dmat * dmat)
    row = lax.broadcasted_iota(jnp.int32, (8, 128), 0)
    col = lax.broadcasted_iota(jnp.int32, (8, 128), 1)
    o = jnp.where(col == 0, kl_loss, ppl)
    sc_ref[...] = jnp.where(row == 0, o, 0.0)


def _tc_p7(pool, cnt, kl2, kl1, pwm, pat):
    return pl.pallas_call(
        _p7_body,
        out_shape=[
            jax.ShapeDtypeStruct((B, EMB), jnp.float32),
            jax.ShapeDtypeStruct((8, 128), jnp.float32),
        ],
    )(pool, cnt, kl2, kl1, pwm, pat)


def _fold(W, b, g, bt):
    s = g * _BNS
    Wt = (W * s[:, None]).T
    bf = b * s + bt
    b8 = jnp.zeros((8, bf.shape[0]), jnp.float32).at[0].set(bf)
    return Wt, b8


def kernel(patient_repr, x, edge_index, batch, params):
    xp = jnp.pad(x, ((0, NP - N), (0, 0)))
    batch_pad = jnp.pad(batch, (0, NP - N), constant_values=PADB)
    batch3 = batch_pad.reshape(NT, 1, TILE)
    src3 = jnp.pad(edge_index[0], (0, EP - E)).reshape(2, NSTG, SCH)
    dst3 = jnp.pad(edge_index[1], (0, EP - E),
                   constant_values=1 << 20).reshape(2, NSTG, SCH)
    csrc, cdstl, cnts = _edge_compact(src3, dst3)

    gin = []
    for lp in params['gin']:
        w1t, b1 = _fold(lp['W1'], lp['b1'], lp['g1'], lp['bt1'])
        w2t, b2 = _fold(lp['W2'], lp['b2'], lp['g2'], lp['bt2'])
        epsv = jnp.broadcast_to((1.0 + lp['eps']).astype(jnp.float32)
                                .reshape(1, 1), (8, 128))
        gin.append((w1t, b1, w2t, b2, epsv))
    vnp = []
    for vp in params['vn']:
        w1t, b1 = _fold(vp['W1'], vp['b1'], vp['g1'], vp['bt1'])
        w2t, b2 = _fold(vp['W2'], vp['b2'], vp['g2'], vp['bt2'])
        vnp.append((w1t, b1, w2t, b2))
    cp = params['comp']
    cw1t, cb1 = _fold(cp['W1'], cp['b1'], cp['g'], cp['bt'])
    cb1x = cb1.at[1, 0].set(cp['b2'][0])
    wp = (jnp.zeros((EMB, 128), jnp.float32)
          .at[:, 0].set(cp['W2'][0])
          .at[:, 1].set(patient_repr[0]))
    pwm = jnp.zeros((EMB, 128), jnp.float32).at[:, 0].set(params['pred']['W'][0])
    pat8 = (jnp.zeros((8, EMB), jnp.float32)
            .at[0].set(patient_repr[0])
            .at[1, 0].set(params['pred']['b'][0]))

    u1 = jax.random.uniform(jax.random.fold_in(jax.random.key(0), 1),
                            (N, 1), jnp.float32)
    u1p = jnp.pad(u1, ((0, NP - N), (0, 127)))
    u2 = jax.random.uniform(jax.random.fold_in(jax.random.key(0), 2),
                            (N, EMB), jnp.float32)
    u2p = jnp.pad(u2, ((0, NP - N), (0, 0)))

    h = xp
    vn = jnp.zeros((B, EMB), jnp.float32)
    for l in range(NUM_LAYER):
        ha, hb, seg = _tc_pre(h, vn, batch3)
        agg = _edge_segsum(ha, hb, csrc, cdstl, cnts)
        w1t, b1, w2t, b2, epsv = gin[l]
        h = _tc_mlp(ha, hb, agg, w1t, b1, w2t, b2, epsv,
                    final=(l == NUM_LAYER - 1))
        if l < NUM_LAYER - 1:
            vw1t, vb1, vw2t, vb2 = vnp[l]
            vn = _tc_vn(seg, vn, vw1t, vb1, vw2t, vb2)

    pt = _tc_p1(h, cw1t, cb1x, wp)
    stats = _tc_p2(pt)
    static, seg, cnt = _tc_p3(h, pt, stats, batch3)
    var = _tc_p5(static, seg, cnt, batch3)
    pool, kl2, kl1 = _tc_p6(static, pt, u1p, u2p, seg, cnt, var, batch3)
    pooled, sc = _tc_p7(pool, cnt, kl2, kl1, pwm, pat8)

    kl_loss = sc[0, 0].reshape(())
    preserve_rate = stats[0, 2].reshape(())
    patient_pred_loss = sc[0, 1].reshape(())
    return (pooled, kl_loss, preserve_rate, patient_pred_loss)


# speedup vs baseline: 1.2506x; 1.2506x over previous
import functools

import jax
import jax.numpy as jnp
from jax import lax
from jax.experimental import pallas as pl
from jax.experimental.pallas import tpu as pltpu
from jax.experimental.pallas import tpu_sc as plsc

EMB = 256
NUM_LAYER = 5
B = 64
N = 10000
E = 160000

NP = 10240
TILE = 1024
NT = NP // TILE
EP = 163840
ECHUNK = 128
NCHUNKS = EP // (16 * ECHUNK)
TRASH = 10200
PADB = 127
_BNS = float(1.0 + 1e-5) ** -0.5


IG = 20
NIG = NCHUNKS // IG


def _make_edge_segsum():
    mesh = plsc.VectorSubcoreMesh(core_axis_name="c", subcore_axis_name="s")

    @functools.partial(
        pl.kernel,
        mesh=mesh,
        out_type=jax.ShapeDtypeStruct((NP, EMB), jnp.float32),
        scratch_types=[
            pltpu.VMEM((2, IG, ECHUNK), jnp.int32),
            pltpu.VMEM((2, IG, ECHUNK), jnp.int32),
            pltpu.VMEM((2, ECHUNK, 128), jnp.float32),
            pltpu.VMEM_SHARED((NP, 128), jnp.float32),
        ] + [pltpu.SemaphoreType.DMA] * 5,
    )
    def k(ha_hbm, hb_hbm, src_hbm, dst_hbm, out_hbm,
          src_v, dst_v, bufs, sh_agg, sg0, sg1, six, ss0, ss1):
        cid = lax.axis_index("c")
        sid = lax.axis_index("s")
        gsem = (sg0, sg1)
        ssem = (ss0, ss1)

        def idx_issue(ig, s):
            pltpu.async_copy(src_hbm.at[sid, ig], src_v.at[s], six)
            pltpu.async_copy(dst_hbm.at[sid, ig], dst_v.at[s], six)

        def idx_wait(ig, s):
            pltpu.make_async_copy(src_hbm.at[sid, ig], src_v.at[s],
                                  six).wait()
            pltpu.make_async_copy(dst_hbm.at[sid, ig], dst_v.at[s],
                                  six).wait()

        idx_issue(0, 0)

        def zrow(r, _):
            for kk in range(8):
                bufs[0, r, pl.ds(kk * 16, 16)] = jnp.zeros((16,), jnp.float32)
            return 0
        lax.fori_loop(0, ECHUNK, zrow, 0)
        for j in range(5):
            pltpu.sync_copy(bufs.at[0],
                            sh_agg.at[pl.ds(sid * 640 + j * 128, 128)])
        plsc.subcore_barrier()

        def issue(s, j, b, h_hbm):
            pltpu.async_copy(h_hbm.at[src_v.at[s, j]], bufs.at[b], gsem[b])

        def gwait(s, j, b, h_hbm):
            pltpu.make_async_copy(h_hbm.at[src_v.at[s, j]], bufs.at[b],
                                  gsem[b]).wait()

        def run(h_hbm):
            for ig in range(NIG):
                s = ig % 2
                idx_wait(ig, s)
                if ig + 1 < NIG:
                    idx_issue(ig + 1, (ig + 1) % 2)
                issue(s, 0, 0, h_hbm)
                issue(s, 1, 1, h_hbm)

                def pair(j2, _):
                    for b in range(2):
                        j = j2 * 2 + b
                        gwait(s, j, b, h_hbm)
                        pltpu.async_copy(bufs.at[b],
                                        sh_agg.at[dst_v.at[s, j]], ssem[b],
                                        add=True)
                    for b in range(2):
                        j = j2 * 2 + b
                        pltpu.make_async_copy(
                            bufs.at[b], sh_agg.at[dst_v.at[s, j]],
                            ssem[b]).wait()

                        @pl.when(j2 < IG // 2 - 1)
                        def _():
                            issue(s, j + 2, b, h_hbm)
                    return 0
                lax.fori_loop(0, IG // 2, pair, 0)

        @pl.when(cid == 0)
        def _():
            run(ha_hbm)

        @pl.when(cid == 1)
        def _():
            run(hb_hbm)

        plsc.subcore_barrier()
        pltpu.sync_copy(
            sh_agg.at[pl.ds(sid * 640, 640)],
            out_hbm.at[pl.ds(sid * 640, 640), pl.ds(cid * 128, 128)])

    return k


@functools.cache
def _edge_segsum_kernel():
    return _make_edge_segsum()


def _edge_segsum(ha, hb, src, dst):
    return _edge_segsum_kernel()(ha, hb, src, dst)


def _dot(a, b):
    return jnp.dot(a, b, preferred_element_type=jnp.float32)


def _oht(batch_ref):
    bt = batch_ref[0]
    cls = lax.broadcasted_iota(jnp.int32, (B, 1), 0)
    return (cls == bt).astype(jnp.float32)


def _pre_body(h_ref, vn_ref, batch_ref, ha_ref, hb_ref, seg_ref):
    oht = _oht(batch_ref)
    h = h_ref[...]
    vng = lax.dot_general(oht, vn_ref[...], (((0,), (0,)), ((), ())),
                          preferred_element_type=jnp.float32)
    hin = h + vng
    ha_ref[...] = hin[:, :128]
    hb_ref[...] = hin[:, 128:]

    @pl.when(pl.program_id(0) == 0)
    def _():
        seg_ref[...] = jnp.zeros_like(seg_ref)
    seg_ref[...] += _dot(oht, h)


def _tc_pre(h, vn, batch3):
    return pl.pallas_call(
        _pre_body,
        grid=(NT,),
        in_specs=[
            pl.BlockSpec((TILE, EMB), lambda i: (i, 0)),
            pl.BlockSpec((B, EMB), lambda i: (0, 0)),
            pl.BlockSpec((1, 1, TILE), lambda i: (i, 0, 0)),
        ],
        out_specs=[
            pl.BlockSpec((TILE, 128), lambda i: (i, 0)),
            pl.BlockSpec((TILE, 128), lambda i: (i, 0)),
            pl.BlockSpec((B, EMB), lambda i: (0, 0)),
        ],
        out_shape=[
            jax.ShapeDtypeStruct((NP, 128), jnp.float32),
            jax.ShapeDtypeStruct((NP, 128), jnp.float32),
            jax.ShapeDtypeStruct((B, EMB), jnp.float32),
        ],
    )(h, vn, batch3)


def _mlp_body(ha_ref, hb_ref, agg_ref, w1_ref, b1_ref, w2_ref, b2_ref,
              eps_ref, out_ref, *, final):
    hin = jnp.concatenate([ha_ref[...], hb_ref[...]], axis=1)
    z = eps_ref[0, 0] * hin + agg_ref[...]
    a = jnp.maximum(_dot(z, w1_ref[...]) + b1_ref[0:1, :], 0.0)
    o = _dot(a, w2_ref[...]) + b2_ref[0:1, :]
    out_ref[...] = o if final else jnp.maximum(o, 0.0)


def _tc_mlp(ha, hb, agg, w1t, b1, w2t, b2, epsv, final):
    return pl.pallas_call(
        functools.partial(_mlp_body, final=final),
        grid=(NT,),
        in_specs=[
            pl.BlockSpec((TILE, 128), lambda i: (i, 0)),
            pl.BlockSpec((TILE, 128), lambda i: (i, 0)),
            pl.BlockSpec((TILE, EMB), lambda i: (i, 0)),
            pl.BlockSpec((EMB, 2 * EMB), lambda i: (0, 0)),
            pl.BlockSpec((8, 2 * EMB), lambda i: (0, 0)),
            pl.BlockSpec((2 * EMB, EMB), lambda i: (0, 0)),
            pl.BlockSpec((8, EMB), lambda i: (0, 0)),
            pl.BlockSpec((8, 128), lambda i: (0, 0)),
        ],
        out_specs=pl.BlockSpec((TILE, EMB), lambda i: (i, 0)),
        out_shape=jax.ShapeDtypeStruct((NP, EMB), jnp.float32),
    )(ha, hb, agg, w1t, b1, w2t, b2, epsv)


def _vn_body(seg_ref, vn_ref, w1_ref, b1_ref, w2_ref, b2_ref, out_ref):
    tmp = seg_ref[...] + vn_ref[...]
    t = jnp.maximum(_dot(tmp, w1_ref[...]) + b1_ref[0:1, :], 0.0)
    out_ref[...] = jnp.maximum(_dot(t, w2_ref[...]) + b2_ref[0:1, :], 0.0)


def _tc_vn(seg, vn, w1t, b1, w2t, b2):
    return pl.pallas_call(
        _vn_body,
        out_shape=jax.ShapeDtypeStruct((B, EMB), jnp.float32),
    )(seg, vn, w1t, b1, w2t, b2)


def _p1_body(h_ref, w1_ref, cb1_ref, wp_ref, pt_ref):
    i = pl.program_id(0)
    h = h_ref[...]
    hc = jnp.maximum(_dot(h, w1_ref[...]) + cb1_ref[0:1, :], 0.0)
    pm = _dot(hc, wp_ref[...])
    sm = _dot(h, wp_ref[...])
    p = pm[:, 0:1] + cb1_ref[1, 0]
    sim = sm[:, 1:2]
    rows = lax.broadcasted_iota(jnp.int32, (TILE, 1), 0) + i * TILE
    valid = rows < N
    pcol = jnp.where(valid, p, 0.0)
    scol = jnp.where(valid, sim, -1e30)
    pt_ref[...] = jnp.concatenate(
        [pcol, scol, jnp.zeros((TILE, 126), jnp.float32)], axis=1)


def _tc_p1(h5, cw1t, cb1x, wp):
    return pl.pallas_call(
        _p1_body,
        grid=(NT,),
        in_specs=[
            pl.BlockSpec((TILE, EMB), lambda i: (i, 0)),
            pl.BlockSpec((EMB, EMB), lambda i: (0, 0)),
            pl.BlockSpec((8, EMB), lambda i: (0, 0)),
            pl.BlockSpec((EMB, 128), lambda i: (0, 0)),
        ],
        out_specs=pl.BlockSpec((TILE, 128), lambda i: (i, 0)),
        out_shape=jax.ShapeDtypeStruct((NP, 128), jnp.float32),
    )(h5, cw1t, cb1x, wp)


def _p2_body(pt_ref, out_ref):
    p = pt_ref[:, 0:1]
    sim = pt_ref[:, 1:2]
    m = jnp.max(sim)
    s = jnp.sum(jnp.exp(sim - m))
    pres = jnp.sum((p > 0.0).astype(jnp.float32)) / jnp.float32(N)
    row = lax.broadcasted_iota(jnp.int32, (8, 128), 0)
    col = lax.broadcasted_iota(jnp.int32, (8, 128), 1)
    o = jnp.where(col == 0, m, jnp.where(col == 1, s, pres))
    out_ref[...] = jnp.where(row == 0, o, 0.0)


def _tc_p2(pt):
    return pl.pallas_call(
        _p2_body,
        out_shape=jax.ShapeDtypeStruct((8, 128), jnp.float32),
    )(pt)


def _p3_body(h_ref, pt_ref, st_ref, batch_ref, static_ref, seg_ref, cnt_ref):
    oht = _oht(batch_ref)
    h = h_ref[...]
    m = st_ref[0, 0]
    s = st_ref[0, 1]
    w = jnp.exp(pt_ref[:, 1:2] - m) / s
    t = h * w
    nrm = jnp.sqrt(jnp.sum(t * t, axis=1, keepdims=True))
    static = t / jnp.maximum(nrm, 1e-12)
    static_ref[...] = static

    @pl.when(pl.program_id(0) == 0)
    def _():
        seg_ref[...] = jnp.zeros_like(seg_ref)
        cnt_ref[...] = jnp.zeros_like(cnt_ref)
    seg_ref[...] += _dot(oht, static)
    cnt_ref[...] += _dot(oht, jnp.ones((TILE, 128), jnp.float32))


def _tc_p3(h5, pt, stats, batch3):
    return pl.pallas_call(
        _p3_body,
        grid=(NT,),
        in_specs=[
            pl.BlockSpec((TILE, EMB), lambda i: (i, 0)),
            pl.BlockSpec((TILE, 128), lambda i: (i, 0)),
            pl.BlockSpec((8, 128), lambda i: (0, 0)),
            pl.BlockSpec((1, 1, TILE), lambda i: (i, 0, 0)),
        ],
        out_specs=[
            pl.BlockSpec((TILE, EMB), lambda i: (i, 0)),
            pl.BlockSpec((B, EMB), lambda i: (0, 0)),
            pl.BlockSpec((B, 128), lambda i: (0, 0)),
        ],
        out_shape=[
            jax.ShapeDtypeStruct((NP, EMB), jnp.float32),
            jax.ShapeDtypeStruct((B, EMB), jnp.float32),
            jax.ShapeDtypeStruct((B, 128), jnp.float32),
        ],
    )(h5, pt, stats, batch3)


def _p5_body(static_ref, seg_ref, cnt_ref, batch_ref, var_ref):
    oht = _oht(batch_ref)
    cnt = jnp.maximum(cnt_ref[:, 0:1], 1.0)
    seg_mean = seg_ref[...] / cnt
    nm = lax.dot_general(oht, seg_mean, (((0,), (0,)), ((), ())),
                         preferred_element_type=jnp.float32)
    diff = static_ref[...] - nm

    @pl.when(pl.program_id(0) == 0)
    def _():
        var_ref[...] = jnp.zeros_like(var_ref)
    var_ref[...] += _dot(oht, diff * diff)


def _tc_p5(static, seg, cnt, batch3):
    return pl.pallas_call(
        _p5_body,
        grid=(NT,),
        in_specs=[
            pl.BlockSpec((TILE, EMB), lambda i: (i, 0)),
            pl.BlockSpec((B, EMB), lambda i: (0, 0)),
            pl.BlockSpec((B, 128), lambda i: (0, 0)),
            pl.BlockSpec((1, 1, TILE), lambda i: (i, 0, 0)),
        ],
        out_specs=pl.BlockSpec((B, EMB), lambda i: (0, 0)),
        out_shape=jax.ShapeDtypeStruct((B, EMB), jnp.float32),
    )(static, seg, cnt, batch3)


def _p6_body(static_ref, pt_ref, u1_ref, noise_ref, seg_ref, cnt_ref, var_ref,
             batch_ref, pool_ref, kl2_ref, kl1_ref):
    oht = _oht(batch_ref)
    onc = lax.dot_general(oht, jnp.ones((B, 1), jnp.float32),
                          (((0,), (0,)), ((), ())),
                          preferred_element_type=jnp.float32)
    counts = cnt_ref[:, 0:1]
    cnt = jnp.maximum(counts, 1.0)
    seg_mean = seg_ref[...] / cnt
    var = var_ref[...] / jnp.maximum(counts - 1.0, 1.0)
    seg_std = jnp.sqrt(jnp.maximum(var, 0.0))
    dg = functools.partial(lax.dot_general,
                           dimension_numbers=(((0,), (0,)), ((), ())),
                           preferred_element_type=jnp.float32)
    node_mean = dg(oht, seg_mean)
    node_std = dg(oht, seg_std)
    static = static_ref[...]

    bias = 0.0001
    u = u1_ref[:, 0:1]
    eps_ = (bias - (1.0 - bias)) * u + (1.0 - bias)
    gate = jnp.log(eps_) - jnp.log(1.0 - eps_) + pt_ref[:, 0:1]
    lam = jax.nn.sigmoid(gate) * onc
    nm = lam * static + (1.0 - lam) * node_mean
    ns = (1.0 - lam) * node_std
    noisy = nm + noise_ref[...] * ns
    e2 = 1e-07
    kl2t = ((nm - node_mean) / (node_std + e2)) ** 2
    kl1t = jnp.mean((ns / (node_std + e2)) ** 2, axis=1, keepdims=True) * onc

    @pl.when(pl.program_id(0) == 0)
    def _():
        pool_ref[...] = jnp.zeros_like(pool_ref)
        kl2_ref[...] = jnp.zeros_like(kl2_ref)
        kl1_ref[...] = jnp.zeros_like(kl1_ref)
    pool_ref[...] += _dot(oht, noisy)
    kl2_ref[...] += _dot(oht, kl2t)
    kl1_ref[:, 0:1] += _dot(oht, kl1t)


def _tc_p6(static, pt, u1p, noise, seg, cnt, var, batch3):
    return pl.pallas_call(
        _p6_body,
        grid=(NT,),
        in_specs=[
            pl.BlockSpec((TILE, EMB), lambda i: (i, 0)),
            pl.BlockSpec((TILE, 128), lambda i: (i, 0)),
            pl.BlockSpec((TILE, 128), lambda i: (i, 0)),
            pl.BlockSpec((TILE, EMB), lambda i: (i, 0)),
            pl.BlockSpec((B, EMB), lambda i: (0, 0)),
            pl.BlockSpec((B, 128), lambda i: (0, 0)),
            pl.BlockSpec((B, EMB), lambda i: (0, 0)),
            pl.BlockSpec((1, 1, TILE), lambda i: (i, 0, 0)),
        ],
        out_specs=[
            pl.BlockSpec((B, EMB), lambda i: (0, 0)),
            pl.BlockSpec((B, EMB), lambda i: (0, 0)),
            pl.BlockSpec((B, 128), lambda i: (0, 0)),
        ],
        out_shape=[
            jax.ShapeDtypeStruct((B, EMB), jnp.float32),
            jax.ShapeDtypeStruct((B, EMB), jnp.float32),
            jax.ShapeDtypeStruct((B, 128), jnp.float32),
        ],
    )(static, pt, u1p, noise, seg, cnt, var, batch3)


def _p7_body(pool_ref, cnt_ref, kl2_ref, kl1_ref, pw_ref, pat_ref,
             pooled_ref, sc_ref):
    cnt = jnp.maximum(cnt_ref[:, 0:1], 1.0)
    pooled = pool_ref[...] / cnt
    pooled_ref[...] = pooled
    kl_loss = (jnp.sum(kl2_ref[...])
               + EMB * 0.5 * jnp.sum(kl1_ref[:, 0:1])) / (B * EMB)
    pred = _dot(pooled, pw_ref[...])[:, 0:1] + pat_ref[1, 0]
    dmat = pat_ref[0:1, :] - pred
    ppl = jnp.mean(dmat * dmat)
    row = lax.broadcasted_iota(jnp.int32, (8, 128), 0)
    col = lax.broadcasted_iota(jnp.int32, (8, 128), 1)
    o = jnp.where(col == 0, kl_loss, ppl)
    sc_ref[...] = jnp.where(row == 0, o, 0.0)


def _tc_p7(pool, cnt, kl2, kl1, pwm, pat):
    return pl.pallas_call(
        _p7_body,
        out_shape=[
            jax.ShapeDtypeStruct((B, EMB), jnp.float32),
            jax.ShapeDtypeStruct((8, 128), jnp.float32),
        ],
    )(pool, cnt, kl2, kl1, pwm, pat)


def _fold(W, b, g, bt):
    s = g * _BNS
    Wt = (W * s[:, None]).T
    bf = b * s + bt
    b8 = jnp.zeros((8, bf.shape[0]), jnp.float32).at[0].set(bf)
    return Wt, b8


def kernel(patient_repr, x, edge_index, batch, params):
    xp = jnp.pad(x, ((0, NP - N), (0, 0)))
    batch_pad = jnp.pad(batch, (0, NP - N), constant_values=PADB)
    batch3 = batch_pad.reshape(NT, 1, TILE)
    src = jnp.pad(edge_index[0], (0, EP - E)).reshape(16, NIG, IG, ECHUNK)
    dst = jnp.pad(edge_index[1], (0, EP - E),
                  constant_values=TRASH).reshape(16, NIG, IG, ECHUNK)

    gin = []
    for lp in params['gin']:
        w1t, b1 = _fold(lp['W1'], lp['b1'], lp['g1'], lp['bt1'])
        w2t, b2 = _fold(lp['W2'], lp['b2'], lp['g2'], lp['bt2'])
        epsv = jnp.broadcast_to((1.0 + lp['eps']).astype(jnp.float32)
                                .reshape(1, 1), (8, 128))
        gin.append((w1t, b1, w2t, b2, epsv))
    vnp = []
    for vp in params['vn']:
        w1t, b1 = _fold(vp['W1'], vp['b1'], vp['g1'], vp['bt1'])
        w2t, b2 = _fold(vp['W2'], vp['b2'], vp['g2'], vp['bt2'])
        vnp.append((w1t, b1, w2t, b2))
    cp = params['comp']
    cw1t, cb1 = _fold(cp['W1'], cp['b1'], cp['g'], cp['bt'])
    cb1x = cb1.at[1, 0].set(cp['b2'][0])
    wp = (jnp.zeros((EMB, 128), jnp.float32)
          .at[:, 0].set(cp['W2'][0])
          .at[:, 1].set(patient_repr[0]))
    pwm = jnp.zeros((EMB, 128), jnp.float32).at[:, 0].set(params['pred']['W'][0])
    pat8 = (jnp.zeros((8, EMB), jnp.float32)
            .at[0].set(patient_repr[0])
            .at[1, 0].set(params['pred']['b'][0]))

    u1 = jax.random.uniform(jax.random.fold_in(jax.random.key(0), 1),
                            (N, 1), jnp.float32)
    u1p = jnp.pad(u1, ((0, NP - N), (0, 127)))
    u2 = jax.random.uniform(jax.random.fold_in(jax.random.key(0), 2),
                            (N, EMB), jnp.float32)
    u2p = jnp.pad(u2, ((0, NP - N), (0, 0)))

    h = xp
    vn = jnp.zeros((B, EMB), jnp.float32)
    for l in range(NUM_LAYER):
        ha, hb, seg = _tc_pre(h, vn, batch3)
        agg = _edge_segsum(ha, hb, src, dst)
        w1t, b1, w2t, b2, epsv = gin[l]
        h = _tc_mlp(ha, hb, agg, w1t, b1, w2t, b2, epsv,
                    final=(l == NUM_LAYER - 1))
        if l < NUM_LAYER - 1:
            vw1t, vb1, vw2t, vb2 = vnp[l]
            vn = _tc_vn(seg, vn, vw1t, vb1, vw2t, vb2)

    pt = _tc_p1(h, cw1t, cb1x, wp)
    stats = _tc_p2(pt)
    static, seg, cnt = _tc_p3(h, pt, stats, batch3)
    var = _tc_p5(static, seg, cnt, batch3)
    pool, kl2, kl1 = _tc_p6(static, pt, u1p, u2p, seg, cnt, var, batch3)
    pooled, sc = _tc_p7(pool, cnt, kl2, kl1, pwm, pat8)

    kl_loss = sc[0, 0].reshape(())
    preserve_rate = stats[0, 2].reshape(())
    patient_pred_loss = sc[0, 1].reshape(())
    return (pooled, kl_loss, preserve_rate, patient_pred_loss)


# speedup vs baseline: 1.3335x; 1.0663x over previous
import functools

import jax
import jax.numpy as jnp
from jax import lax
from jax.experimental import pallas as pl
from jax.experimental.pallas import tpu as pltpu
from jax.experimental.pallas import tpu_sc as plsc

EMB = 256
NUM_LAYER = 5
B = 64
N = 10000
E = 160000

NP = 10240
TILE = 1024
NT = NP // TILE
EP = 163840
ECHUNK = 128
NCHUNKS = EP // (16 * ECHUNK)
TRASH = 10200
PADB = 127
_BNS = float(1.0 + 1e-5) ** -0.5


IG = 20
NIG = NCHUNKS // IG


def _make_edge_segsum():
    mesh = plsc.VectorSubcoreMesh(core_axis_name="c", subcore_axis_name="s")

    @functools.partial(
        pl.kernel,
        mesh=mesh,
        out_type=jax.ShapeDtypeStruct((NP, EMB), jnp.float32),
        scratch_types=[
            pltpu.VMEM((2, IG, ECHUNK), jnp.int32),
            pltpu.VMEM((2, IG, ECHUNK), jnp.int32),
            pltpu.VMEM((2, ECHUNK, 128), jnp.float32),
            pltpu.VMEM_SHARED((NP, 128), jnp.float32),
        ] + [pltpu.SemaphoreType.DMA] * 3,
    )
    def k(ha_hbm, hb_hbm, src_hbm, dst_hbm, out_hbm,
          src_v, dst_v, bufs, sh_agg, sg0, sg1, six):
        cid = lax.axis_index("c")
        sid = lax.axis_index("s")
        gsem = (sg0, sg1)

        def idx_issue(ig, s):
            pltpu.async_copy(src_hbm.at[sid, ig], src_v.at[s], six)
            pltpu.async_copy(dst_hbm.at[sid, ig], dst_v.at[s], six)

        def idx_wait(ig, s):
            pltpu.make_async_copy(src_hbm.at[sid, ig], src_v.at[s],
                                  six).wait()
            pltpu.make_async_copy(dst_hbm.at[sid, ig], dst_v.at[s],
                                  six).wait()

        idx_issue(0, 0)

        def zrow(r, _):
            for kk in range(8):
                bufs[0, r, pl.ds(kk * 16, 16)] = jnp.zeros((16,), jnp.float32)
            return 0
        lax.fori_loop(0, ECHUNK, zrow, 0)
        for j in range(5):
            pltpu.sync_copy(bufs.at[0],
                            sh_agg.at[pl.ds(sid * 640 + j * 128, 128)])
        plsc.subcore_barrier()

        def issue(s, j, b, h_hbm):
            pltpu.async_copy(h_hbm.at[src_v.at[s, j]], bufs.at[b], gsem[b])

        def gwait(s, j, b, h_hbm):
            pltpu.make_async_copy(h_hbm.at[src_v.at[s, j]], bufs.at[b],
                                  gsem[b]).wait()

        def run(h_hbm):
            for ig in range(NIG):
                s = ig % 2
                idx_wait(ig, s)
                if ig + 1 < NIG:
                    idx_issue(ig + 1, (ig + 1) % 2)
                issue(s, 0, 0, h_hbm)
                issue(s, 1, 1, h_hbm)

                def pair(j2, _):
                    for b in range(2):
                        j = j2 * 2 + b
                        gwait(s, j, b, h_hbm)
                        pltpu.sync_copy(bufs.at[b],
                                        sh_agg.at[dst_v.at[s, j]], add=True)

                        @pl.when(j2 < IG // 2 - 1)
                        def _():
                            issue(s, j + 2, b, h_hbm)
                    return 0
                lax.fori_loop(0, IG // 2, pair, 0)

        @pl.when(cid == 0)
        def _():
            run(ha_hbm)

        @pl.when(cid == 1)
        def _():
            run(hb_hbm)

        plsc.subcore_barrier()
        pltpu.sync_copy(
            sh_agg.at[pl.ds(sid * 640, 640)],
            out_hbm.at[pl.ds(sid * 640, 640), pl.ds(cid * 128, 128)])

    return k


@functools.cache
def _edge_segsum_kernel():
    return _make_edge_segsum()


def _edge_segsum(ha, hb, src, dst):
    return _edge_segsum_kernel()(ha, hb, src, dst)


def _dot(a, b):
    return jnp.dot(a, b, preferred_element_type=jnp.float32)


def _oht(batch_ref):
    bt = batch_ref[0]
    cls = lax.broadcasted_iota(jnp.int32, (B, 1), 0)
    return (cls == bt).astype(jnp.float32)


def _pre_body(h_ref, vn_ref, batch_ref, ha_ref, hb_ref, seg_ref):
    oht = _oht(batch_ref)
    h = h_ref[...]
    vng = lax.dot_general(oht, vn_ref[...], (((0,), (0,)), ((), ())),
                          preferred_element_type=jnp.float32)
    hin = h + vng
    ha_ref[...] = hin[:, :128]
    hb_ref[...] = hin[:, 128:]

    @pl.when(pl.program_id(0) == 0)
    def _():
        seg_ref[...] = jnp.zeros_like(seg_ref)
    seg_ref[...] += _dot(oht, h)


def _tc_pre(h, vn, batch3):
    return pl.pallas_call(
        _pre_body,
        grid=(NT,),
        in_specs=[
            pl.BlockSpec((TILE, EMB), lambda i: (i, 0)),
            pl.BlockSpec((B, EMB), lambda i: (0, 0)),
            pl.BlockSpec((1, 1, TILE), lambda i: (i, 0, 0)),
        ],
        out_specs=[
            pl.BlockSpec((TILE, 128), lambda i: (i, 0)),
            pl.BlockSpec((TILE, 128), lambda i: (i, 0)),
            pl.BlockSpec((B, EMB), lambda i: (0, 0)),
        ],
        out_shape=[
            jax.ShapeDtypeStruct((NP, 128), jnp.float32),
            jax.ShapeDtypeStruct((NP, 128), jnp.float32),
            jax.ShapeDtypeStruct((B, EMB), jnp.float32),
        ],
    )(h, vn, batch3)


def _mlp_body(ha_ref, hb_ref, agg_ref, w1_ref, b1_ref, w2_ref, b2_ref,
              eps_ref, out_ref, *, final):
    hin = jnp.concatenate([ha_ref[...], hb_ref[...]], axis=1)
    z = eps_ref[0, 0] * hin + agg_ref[...]
    a = jnp.maximum(_dot(z, w1_ref[...]) + b1_ref[0:1, :], 0.0)
    o = _dot(a, w2_ref[...]) + b2_ref[0:1, :]
    out_ref[...] = o if final else jnp.maximum(o, 0.0)


def _tc_mlp(ha, hb, agg, w1t, b1, w2t, b2, epsv, final):
    return pl.pallas_call(
        functools.partial(_mlp_body, final=final),
        grid=(NT,),
        in_specs=[
            pl.BlockSpec((TILE, 128), lambda i: (i, 0)),
            pl.BlockSpec((TILE, 128), lambda i: (i, 0)),
            pl.BlockSpec((TILE, EMB), lambda i: (i, 0)),
            pl.BlockSpec((EMB, 2 * EMB), lambda i: (0, 0)),
            pl.BlockSpec((8, 2 * EMB), lambda i: (0, 0)),
            pl.BlockSpec((2 * EMB, EMB), lambda i: (0, 0)),
            pl.BlockSpec((8, EMB), lambda i: (0, 0)),
            pl.BlockSpec((8, 128), lambda i: (0, 0)),
        ],
        out_specs=pl.BlockSpec((TILE, EMB), lambda i: (i, 0)),
        out_shape=jax.ShapeDtypeStruct((NP, EMB), jnp.float32),
    )(ha, hb, agg, w1t, b1, w2t, b2, epsv)


def _vn_body(seg_ref, vn_ref, w1_ref, b1_ref, w2_ref, b2_ref, out_ref):
    tmp = seg_ref[...] + vn_ref[...]
    t = jnp.maximum(_dot(tmp, w1_ref[...]) + b1_ref[0:1, :], 0.0)
    out_ref[...] = jnp.maximum(_dot(t, w2_ref[...]) + b2_ref[0:1, :], 0.0)


def _tc_vn(seg, vn, w1t, b1, w2t, b2):
    return pl.pallas_call(
        _vn_body,
        out_shape=jax.ShapeDtypeStruct((B, EMB), jnp.float32),
    )(seg, vn, w1t, b1, w2t, b2)


def _p1_body(h_ref, w1_ref, cb1_ref, wp_ref, pt_ref):
    i = pl.program_id(0)
    h = h_ref[...]
    hc = jnp.maximum(_dot(h, w1_ref[...]) + cb1_ref[0:1, :], 0.0)
    pm = _dot(hc, wp_ref[...])
    sm = _dot(h, wp_ref[...])
    p = pm[:, 0:1] + cb1_ref[1, 0]
    sim = sm[:, 1:2]
    rows = lax.broadcasted_iota(jnp.int32, (TILE, 1), 0) + i * TILE
    valid = rows < N
    pcol = jnp.where(valid, p, 0.0)
    scol = jnp.where(valid, sim, -1e30)
    pt_ref[...] = jnp.concatenate(
        [pcol, scol, jnp.zeros((TILE, 126), jnp.float32)], axis=1)


def _tc_p1(h5, cw1t, cb1x, wp):
    return pl.pallas_call(
        _p1_body,
        grid=(NT,),
        in_specs=[
            pl.BlockSpec((TILE, EMB), lambda i: (i, 0)),
            pl.BlockSpec((EMB, EMB), lambda i: (0, 0)),
            pl.BlockSpec((8, EMB), lambda i: (0, 0)),
            pl.BlockSpec((EMB, 128), lambda i: (0, 0)),
        ],
        out_specs=pl.BlockSpec((TILE, 128), lambda i: (i, 0)),
        out_shape=jax.ShapeDtypeStruct((NP, 128), jnp.float32),
    )(h5, cw1t, cb1x, wp)


def _p2_body(pt_ref, out_ref):
    p = pt_ref[:, 0:1]
    sim = pt_ref[:, 1:2]
    m = jnp.max(sim)
    s = jnp.sum(jnp.exp(sim - m))
    pres = jnp.sum((p > 0.0).astype(jnp.float32)) / jnp.float32(N)
    row = lax.broadcasted_iota(jnp.int32, (8, 128), 0)
    col = lax.broadcasted_iota(jnp.int32, (8, 128), 1)
    o = jnp.where(col == 0, m, jnp.where(col == 1, s, pres))
    out_ref[...] = jnp.where(row == 0, o, 0.0)


def _tc_p2(pt):
    return pl.pallas_call(
        _p2_body,
        out_shape=jax.ShapeDtypeStruct((8, 128), jnp.float32),
    )(pt)


def _p3_body(h_ref, pt_ref, st_ref, batch_ref, static_ref, seg_ref, cnt_ref):
    oht = _oht(batch_ref)
    h = h_ref[...]
    m = st_ref[0, 0]
    s = st_ref[0, 1]
    w = jnp.exp(pt_ref[:, 1:2] - m) / s
    t = h * w
    nrm = jnp.sqrt(jnp.sum(t * t, axis=1, keepdims=True))
    static = t / jnp.maximum(nrm, 1e-12)
    static_ref[...] = static

    @pl.when(pl.program_id(0) == 0)
    def _():
        seg_ref[...] = jnp.zeros_like(seg_ref)
        cnt_ref[...] = jnp.zeros_like(cnt_ref)
    seg_ref[...] += _dot(oht, static)
    cnt_ref[...] += _dot(oht, jnp.ones((TILE, 128), jnp.float32))


def _tc_p3(h5, pt, stats, batch3):
    return pl.pallas_call(
        _p3_body,
        grid=(NT,),
        in_specs=[
            pl.BlockSpec((TILE, EMB), lambda i: (i, 0)),
            pl.BlockSpec((TILE, 128), lambda i: (i, 0)),
            pl.BlockSpec((8, 128), lambda i: (0, 0)),
            pl.BlockSpec((1, 1, TILE), lambda i: (i, 0, 0)),
        ],
        out_specs=[
            pl.BlockSpec((TILE, EMB), lambda i: (i, 0)),
            pl.BlockSpec((B, EMB), lambda i: (0, 0)),
            pl.BlockSpec((B, 128), lambda i: (0, 0)),
        ],
        out_shape=[
            jax.ShapeDtypeStruct((NP, EMB), jnp.float32),
            jax.ShapeDtypeStruct((B, EMB), jnp.float32),
            jax.ShapeDtypeStruct((B, 128), jnp.float32),
        ],
    )(h5, pt, stats, batch3)


def _p5_body(static_ref, seg_ref, cnt_ref, batch_ref, var_ref):
    oht = _oht(batch_ref)
    cnt = jnp.maximum(cnt_ref[:, 0:1], 1.0)
    seg_mean = seg_ref[...] / cnt
    nm = lax.dot_general(oht, seg_mean, (((0,), (0,)), ((), ())),
                         preferred_element_type=jnp.float32)
    diff = static_ref[...] - nm

    @pl.when(pl.program_id(0) == 0)
    def _():
        var_ref[...] = jnp.zeros_like(var_ref)
    var_ref[...] += _dot(oht, diff * diff)


def _tc_p5(static, seg, cnt, batch3):
    return pl.pallas_call(
        _p5_body,
        grid=(NT,),
        in_specs=[
            pl.BlockSpec((TILE, EMB), lambda i: (i, 0)),
            pl.BlockSpec((B, EMB), lambda i: (0, 0)),
            pl.BlockSpec((B, 128), lambda i: (0, 0)),
            pl.BlockSpec((1, 1, TILE), lambda i: (i, 0, 0)),
        ],
        out_specs=pl.BlockSpec((B, EMB), lambda i: (0, 0)),
        out_shape=jax.ShapeDtypeStruct((B, EMB), jnp.float32),
    )(static, seg, cnt, batch3)


def _p6_body(static_ref, pt_ref, u1_ref, noise_ref, seg_ref, cnt_ref, var_ref,
             batch_ref, pool_ref, kl2_ref, kl1_ref):
    oht = _oht(batch_ref)
    onc = lax.dot_general(oht, jnp.ones((B, 1), jnp.float32),
                          (((0,), (0,)), ((), ())),
                          preferred_element_type=jnp.float32)
    counts = cnt_ref[:, 0:1]
    cnt = jnp.maximum(counts, 1.0)
    seg_mean = seg_ref[...] / cnt
    var = var_ref[...] / jnp.maximum(counts - 1.0, 1.0)
    seg_std = jnp.sqrt(jnp.maximum(var, 0.0))
    dg = functools.partial(lax.dot_general,
                           dimension_numbers=(((0,), (0,)), ((), ())),
                           preferred_element_type=jnp.float32)
    node_mean = dg(oht, seg_mean)
    node_std = dg(oht, seg_std)
    static = static_ref[...]

    bias = 0.0001
    u = u1_ref[:, 0:1]
    eps_ = (bias - (1.0 - bias)) * u + (1.0 - bias)
    gate = jnp.log(eps_) - jnp.log(1.0 - eps_) + pt_ref[:, 0:1]
    lam = jax.nn.sigmoid(gate) * onc
    nm = lam * static + (1.0 - lam) * node_mean
    ns = (1.0 - lam) * node_std
    noisy = nm + noise_ref[...] * ns
    e2 = 1e-07
    kl2t = ((nm - node_mean) / (node_std + e2)) ** 2
    kl1t = jnp.mean((ns / (node_std + e2)) ** 2, axis=1, keepdims=True) * onc

    @pl.when(pl.program_id(0) == 0)
    def _():
        pool_ref[...] = jnp.zeros_like(pool_ref)
        kl2_ref[...] = jnp.zeros_like(kl2_ref)
        kl1_ref[...] = jnp.zeros_like(kl1_ref)
    pool_ref[...] += _dot(oht, noisy)
    kl2_ref[...] += _dot(oht, kl2t)
    kl1_ref[:, 0:1] += _dot(oht, kl1t)


def _tc_p6(static, pt, u1p, noise, seg, cnt, var, batch3):
    return pl.pallas_call(
        _p6_body,
        grid=(NT,),
        in_specs=[
            pl.BlockSpec((TILE, EMB), lambda i: (i, 0)),
            pl.BlockSpec((TILE, 128), lambda i: (i, 0)),
            pl.BlockSpec((TILE, 128), lambda i: (i, 0)),
            pl.BlockSpec((TILE, EMB), lambda i: (i, 0)),
            pl.BlockSpec((B, EMB), lambda i: (0, 0)),
            pl.BlockSpec((B, 128), lambda i: (0, 0)),
            pl.BlockSpec((B, EMB), lambda i: (0, 0)),
            pl.BlockSpec((1, 1, TILE), lambda i: (i, 0, 0)),
        ],
        out_specs=[
            pl.BlockSpec((B, EMB), lambda i: (0, 0)),
            pl.BlockSpec((B, EMB), lambda i: (0, 0)),
            pl.BlockSpec((B, 128), lambda i: (0, 0)),
        ],
        out_shape=[
            jax.ShapeDtypeStruct((B, EMB), jnp.float32),
            jax.ShapeDtypeStruct((B, EMB), jnp.float32),
            jax.ShapeDtypeStruct((B, 128), jnp.float32),
        ],
    )(static, pt, u1p, noise, seg, cnt, var, batch3)


def _p7_body(pool_ref, cnt_ref, kl2_ref, kl1_ref, pw_ref, pat_ref,
             pooled_ref, sc_ref):
    cnt = jnp.maximum(cnt_ref[:, 0:1], 1.0)
    pooled = pool_ref[...] / cnt
    pooled_ref[...] = pooled
    kl_loss = (jnp.sum(kl2_ref[...])
               + EMB * 0.5 * jnp.sum(kl1_ref[:, 0:1])) / (B * EMB)
    pred = _dot(pooled, pw_ref[...])[:, 0:1] + pat_ref[1, 0]
    dmat = pat_ref[0:1, :] - pred
    ppl = jnp.mean(dmat * dmat)
    row = lax.broadcasted_iota(jnp.int32, (8, 128), 0)
    col = lax.broadcasted_iota(jnp.int32, (8, 128), 1)
    o = jnp.where(col == 0, kl_loss, ppl)
    sc_ref[...] = jnp.where(row == 0, o, 0.0)


def _tc_p7(pool, cnt, kl2, kl1, pwm, pat):
    return pl.pallas_call(
        _p7_body,
        out_shape=[
            jax.ShapeDtypeStruct((B, EMB), jnp.float32),
            jax.ShapeDtypeStruct((8, 128), jnp.float32),
        ],
    )(pool, cnt, kl2, kl1, pwm, pat)


def _fold(W, b, g, bt):
    s = g * _BNS
    Wt = (W * s[:, None]).T
    bf = b * s + bt
    b8 = jnp.zeros((8, bf.shape[0]), jnp.float32).at[0].set(bf)
    return Wt, b8


def kernel(patient_repr, x, edge_index, batch, params):
    xp = jnp.pad(x, ((0, NP - N), (0, 0)))
    batch_pad = jnp.pad(batch, (0, NP - N), constant_values=PADB)
    batch3 = batch_pad.reshape(NT, 1, TILE)
    src = jnp.pad(edge_index[0], (0, EP - E)).reshape(16, NIG, IG, ECHUNK)
    dst = jnp.pad(edge_index[1], (0, EP - E),
                  constant_values=TRASH).reshape(16, NIG, IG, ECHUNK)

    gin = []
    for lp in params['gin']:
        w1t, b1 = _fold(lp['W1'], lp['b1'], lp['g1'], lp['bt1'])
        w2t, b2 = _fold(lp['W2'], lp['b2'], lp['g2'], lp['bt2'])
        epsv = jnp.broadcast_to((1.0 + lp['eps']).astype(jnp.float32)
                                .reshape(1, 1), (8, 128))
        gin.append((w1t, b1, w2t, b2, epsv))
    vnp = []
    for vp in params['vn']:
        w1t, b1 = _fold(vp['W1'], vp['b1'], vp['g1'], vp['bt1'])
        w2t, b2 = _fold(vp['W2'], vp['b2'], vp['g2'], vp['bt2'])
        vnp.append((w1t, b1, w2t, b2))
    cp = params['comp']
    cw1t, cb1 = _fold(cp['W1'], cp['b1'], cp['g'], cp['bt'])
    cb1x = cb1.at[1, 0].set(cp['b2'][0])
    wp = (jnp.zeros((EMB, 128), jnp.float32)
          .at[:, 0].set(cp['W2'][0])
          .at[:, 1].set(patient_repr[0]))
    pwm = jnp.zeros((EMB, 128), jnp.float32).at[:, 0].set(params['pred']['W'][0])
    pat8 = (jnp.zeros((8, EMB), jnp.float32)
            .at[0].set(patient_repr[0])
            .at[1, 0].set(params['pred']['b'][0]))

    u1 = jax.random.uniform(jax.random.fold_in(jax.random.key(0), 1),
                            (N, 1), jnp.float32)
    u1p = jnp.pad(u1, ((0, NP - N), (0, 127)))
    u2 = jax.random.uniform(jax.random.fold_in(jax.random.key(0), 2),
                            (N, EMB), jnp.float32)
    u2p = jnp.pad(u2, ((0, NP - N), (0, 0)))

    h = xp
    vn = jnp.zeros((B, EMB), jnp.float32)
    for l in range(NUM_LAYER):
        ha, hb, seg = _tc_pre(h, vn, batch3)
        agg = _edge_segsum(ha, hb, src, dst)
        w1t, b1, w2t, b2, epsv = gin[l]
        h = _tc_mlp(ha, hb, agg, w1t, b1, w2t, b2, epsv,
                    final=(l == NUM_LAYER - 1))
        if l < NUM_LAYER - 1:
            vw1t, vb1, vw2t, vb2 = vnp[l]
            vn = _tc_vn(seg, vn, vw1t, vb1, vw2t, vb2)

    pt = _tc_p1(h, cw1t, cb1x, wp)
    stats = _tc_p2(pt)
    static, seg, cnt = _tc_p3(h, pt, stats, batch3)
    var = _tc_p5(static, seg, cnt, batch3)
    pool, kl2, kl1 = _tc_p6(static, pt, u1p, u2p, seg, cnt, var, batch3)
    pooled, sc = _tc_p7(pool, cnt, kl2, kl1, pwm, pat8)

    kl_loss = sc[0, 0].reshape(())
    preserve_rate = stats[0, 2].reshape(())
    patient_pred_loss = sc[0, 1].reshape(())
    return (pooled, kl_loss, preserve_rate, patient_pred_loss)


# speedup vs baseline: 2.9516x; 2.2134x over previous
import functools

import jax
import jax.numpy as jnp
from jax import lax
from jax.experimental import pallas as pl
from jax.experimental.pallas import tpu as pltpu
from jax.experimental.pallas import tpu_sc as plsc

EMB = 256
NUM_LAYER = 5
B = 64
N = 10000
E = 160000

NP = 10240
TILE = 1024
NT = NP // TILE
EP = 163840
ECHUNK = 128
NCHUNKS = EP // (16 * ECHUNK)
TRASH = 10200
PADB = 127
_BNS = float(1.0 + 1e-5) ** -0.5


IG = 20
NIG = NCHUNKS // IG


def _make_edge_segsum():
    mesh = plsc.VectorSubcoreMesh(core_axis_name="c", subcore_axis_name="s")

    @functools.partial(
        pl.kernel,
        mesh=mesh,
        out_type=jax.ShapeDtypeStruct((NP, EMB), jnp.float32),
        scratch_types=[
            pltpu.VMEM((2, IG, ECHUNK), jnp.int32),
            pltpu.VMEM((2, IG, ECHUNK), jnp.int32),
            pltpu.VMEM((2, ECHUNK, 128), jnp.float32),
            pltpu.VMEM_SHARED((NP, 128), jnp.float32),
        ] + [pltpu.SemaphoreType.DMA] * 3,
    )
    def k(ha_hbm, hb_hbm, src_hbm, dst_hbm, out_hbm,
          src_v, dst_v, bufs, sh_agg, sg0, sg1, six):
        cid = lax.axis_index("c")
        sid = lax.axis_index("s")
        gsem = (sg0, sg1)

        def idx_issue(ig, s):
            pltpu.async_copy(src_hbm.at[sid, ig], src_v.at[s], six)
            pltpu.async_copy(dst_hbm.at[sid, ig], dst_v.at[s], six)

        def idx_wait(ig, s):
            pltpu.make_async_copy(src_hbm.at[sid, ig], src_v.at[s],
                                  six).wait()
            pltpu.make_async_copy(dst_hbm.at[sid, ig], dst_v.at[s],
                                  six).wait()

        idx_issue(0, 0)

        def zrow(r, _):
            for kk in range(8):
                bufs[0, r, pl.ds(kk * 16, 16)] = jnp.zeros((16,), jnp.float32)
            return 0
        lax.fori_loop(0, ECHUNK, zrow, 0)
        for j in range(5):
            pltpu.sync_copy(bufs.at[0],
                            sh_agg.at[pl.ds(sid * 640 + j * 128, 128)])
        plsc.subcore_barrier()

        def issue(s, j, b, h_hbm):
            pltpu.async_copy(h_hbm.at[src_v.at[s, j]], bufs.at[b], gsem[b])

        def gwait(s, j, b, h_hbm):
            pltpu.make_async_copy(h_hbm.at[src_v.at[s, j]], bufs.at[b],
                                  gsem[b]).wait()

        def run(h_hbm):
            for ig in range(NIG):
                s = ig % 2
                idx_wait(ig, s)
                if ig + 1 < NIG:
                    idx_issue(ig + 1, (ig + 1) % 2)
                issue(s, 0, 0, h_hbm)
                issue(s, 1, 1, h_hbm)

                def pair(j2, _):
                    for b in range(2):
                        j = j2 * 2 + b
                        gwait(s, j, b, h_hbm)
                        pltpu.sync_copy(bufs.at[b],
                                        sh_agg.at[dst_v.at[s, j]], add=True)

                        @pl.when(j2 < IG // 2 - 1)
                        def _():
                            issue(s, j + 2, b, h_hbm)
                    return 0
                lax.fori_loop(0, IG // 2, pair, 0)

        @pl.when(cid == 0)
        def _():
            run(ha_hbm)

        @pl.when(cid == 1)
        def _():
            run(hb_hbm)

        plsc.subcore_barrier()
        pltpu.sync_copy(
            sh_agg.at[pl.ds(sid * 640, 640)],
            out_hbm.at[pl.ds(sid * 640, 640), pl.ds(cid * 128, 128)])

    return k


@functools.cache
def _edge_segsum_kernel():
    return _make_edge_segsum()


def _edge_segsum(ha, hb, src, dst):
    return _edge_segsum_kernel()(ha, hb, src, dst)


def _dot(a, b):
    return jnp.dot(a, b, preferred_element_type=jnp.float32)


def _oht(batch_ref):
    bt = batch_ref[0]
    cls = lax.broadcasted_iota(jnp.int32, (B, 1), 0)
    return (cls == bt).astype(jnp.float32)


def _pre_body(h_ref, vn_ref, batch_ref, ha_ref, hb_ref, seg_ref):
    oht = _oht(batch_ref)
    h = h_ref[...]
    vng = lax.dot_general(oht, vn_ref[...], (((0,), (0,)), ((), ())),
                          preferred_element_type=jnp.float32)
    hin = h + vng
    ha_ref[...] = hin[:, :128]
    hb_ref[...] = hin[:, 128:]

    @pl.when(pl.program_id(0) == 0)
    def _():
        seg_ref[...] = jnp.zeros_like(seg_ref)
    seg_ref[...] += _dot(oht, h)


def _tc_pre(h, vn, batch3):
    return pl.pallas_call(
        _pre_body,
        grid=(NT,),
        in_specs=[
            pl.BlockSpec((TILE, EMB), lambda i: (i, 0)),
            pl.BlockSpec((B, EMB), lambda i: (0, 0)),
            pl.BlockSpec((1, 1, TILE), lambda i: (i, 0, 0)),
        ],
        out_specs=[
            pl.BlockSpec((TILE, 128), lambda i: (i, 0)),
            pl.BlockSpec((TILE, 128), lambda i: (i, 0)),
            pl.BlockSpec((B, EMB), lambda i: (0, 0)),
        ],
        out_shape=[
            jax.ShapeDtypeStruct((NP, 128), jnp.float32),
            jax.ShapeDtypeStruct((NP, 128), jnp.float32),
            jax.ShapeDtypeStruct((B, EMB), jnp.float32),
        ],
    )(h, vn, batch3)


def _mlp_body(ha_ref, hb_ref, agg_ref, w1_ref, b1_ref, w2_ref, b2_ref,
              eps_ref, out_ref, *, final):
    hin = jnp.concatenate([ha_ref[...], hb_ref[...]], axis=1)
    z = eps_ref[0, 0] * hin + agg_ref[...]
    a = jnp.maximum(_dot(z, w1_ref[...]) + b1_ref[0:1, :], 0.0)
    o = _dot(a, w2_ref[...]) + b2_ref[0:1, :]
    out_ref[...] = o if final else jnp.maximum(o, 0.0)


def _tc_mlp(ha, hb, agg, w1t, b1, w2t, b2, epsv, final):
    return pl.pallas_call(
        functools.partial(_mlp_body, final=final),
        grid=(NT,),
        in_specs=[
            pl.BlockSpec((TILE, 128), lambda i: (i, 0)),
            pl.BlockSpec((TILE, 128), lambda i: (i, 0)),
            pl.BlockSpec((TILE, EMB), lambda i: (i, 0)),
            pl.BlockSpec((EMB, 2 * EMB), lambda i: (0, 0)),
            pl.BlockSpec((8, 2 * EMB), lambda i: (0, 0)),
            pl.BlockSpec((2 * EMB, EMB), lambda i: (0, 0)),
            pl.BlockSpec((8, EMB), lambda i: (0, 0)),
            pl.BlockSpec((8, 128), lambda i: (0, 0)),
        ],
        out_specs=pl.BlockSpec((TILE, EMB), lambda i: (i, 0)),
        out_shape=jax.ShapeDtypeStruct((NP, EMB), jnp.float32),
    )(ha, hb, agg, w1t, b1, w2t, b2, epsv)


def _vn_body(seg_ref, vn_ref, w1_ref, b1_ref, w2_ref, b2_ref, out_ref):
    tmp = seg_ref[...] + vn_ref[...]
    t = jnp.maximum(_dot(tmp, w1_ref[...]) + b1_ref[0:1, :], 0.0)
    out_ref[...] = jnp.maximum(_dot(t, w2_ref[...]) + b2_ref[0:1, :], 0.0)


def _tc_vn(seg, vn, w1t, b1, w2t, b2):
    return pl.pallas_call(
        _vn_body,
        out_shape=jax.ShapeDtypeStruct((B, EMB), jnp.float32),
    )(seg, vn, w1t, b1, w2t, b2)


def _p1_body(h_ref, w1_ref, cb1_ref, wp_ref, pt_ref):
    i = pl.program_id(0)
    h = h_ref[...]
    hc = jnp.maximum(_dot(h, w1_ref[...]) + cb1_ref[0:1, :], 0.0)
    pm = _dot(hc, wp_ref[...])
    sm = _dot(h, wp_ref[...])
    p = pm[:, 0:1] + cb1_ref[1, 0]
    sim = sm[:, 1:2]
    rows = lax.broadcasted_iota(jnp.int32, (TILE, 1), 0) + i * TILE
    valid = rows < N
    pcol = jnp.where(valid, p, 0.0)
    scol = jnp.where(valid, sim, -1e30)
    pt_ref[...] = jnp.concatenate(
        [pcol, scol, jnp.zeros((TILE, 126), jnp.float32)], axis=1)


def _tc_p1(h5, cw1t, cb1x, wp):
    return pl.pallas_call(
        _p1_body,
        grid=(NT,),
        in_specs=[
            pl.BlockSpec((TILE, EMB), lambda i: (i, 0)),
            pl.BlockSpec((EMB, EMB), lambda i: (0, 0)),
            pl.BlockSpec((8, EMB), lambda i: (0, 0)),
            pl.BlockSpec((EMB, 128), lambda i: (0, 0)),
        ],
        out_specs=pl.BlockSpec((TILE, 128), lambda i: (i, 0)),
        out_shape=jax.ShapeDtypeStruct((NP, 128), jnp.float32),
    )(h5, cw1t, cb1x, wp)


def _p2_body(pt_ref, out_ref):
    p = pt_ref[:, 0:1]
    sim = pt_ref[:, 1:2]
    m = jnp.max(sim)
    s = jnp.sum(jnp.exp(sim - m))
    pres = jnp.sum((p > 0.0).astype(jnp.float32)) / jnp.float32(N)
    row = lax.broadcasted_iota(jnp.int32, (8, 128), 0)
    col = lax.broadcasted_iota(jnp.int32, (8, 128), 1)
    o = jnp.where(col == 0, m, jnp.where(col == 1, s, pres))
    out_ref[...] = jnp.where(row == 0, o, 0.0)


def _tc_p2(pt):
    return pl.pallas_call(
        _p2_body,
        out_shape=jax.ShapeDtypeStruct((8, 128), jnp.float32),
    )(pt)


def _p3_body(h_ref, pt_ref, st_ref, batch_ref, static_ref, seg_ref, cnt_ref):
    oht = _oht(batch_ref)
    h = h_ref[...]
    m = st_ref[0, 0]
    s = st_ref[0, 1]
    w = jnp.exp(pt_ref[:, 1:2] - m) / s
    t = h * w
    nrm = jnp.sqrt(jnp.sum(t * t, axis=1, keepdims=True))
    static = t / jnp.maximum(nrm, 1e-12)
    static_ref[...] = static

    @pl.when(pl.program_id(0) == 0)
    def _():
        seg_ref[...] = jnp.zeros_like(seg_ref)
        cnt_ref[...] = jnp.zeros_like(cnt_ref)
    seg_ref[...] += _dot(oht, static)
    cnt_ref[...] += _dot(oht, jnp.ones((TILE, 128), jnp.float32))


def _tc_p3(h5, pt, stats, batch3):
    return pl.pallas_call(
        _p3_body,
        grid=(NT,),
        in_specs=[
            pl.BlockSpec((TILE, EMB), lambda i: (i, 0)),
            pl.BlockSpec((TILE, 128), lambda i: (i, 0)),
            pl.BlockSpec((8, 128), lambda i: (0, 0)),
            pl.BlockSpec((1, 1, TILE), lambda i: (i, 0, 0)),
        ],
        out_specs=[
            pl.BlockSpec((TILE, EMB), lambda i: (i, 0)),
            pl.BlockSpec((B, EMB), lambda i: (0, 0)),
            pl.BlockSpec((B, 128), lambda i: (0, 0)),
        ],
        out_shape=[
            jax.ShapeDtypeStruct((NP, EMB), jnp.float32),
            jax.ShapeDtypeStruct((B, EMB), jnp.float32),
            jax.ShapeDtypeStruct((B, 128), jnp.float32),
        ],
    )(h5, pt, stats, batch3)


def _p5_body(static_ref, seg_ref, cnt_ref, batch_ref, var_ref):
    oht = _oht(batch_ref)
    cnt = jnp.maximum(cnt_ref[:, 0:1], 1.0)
    seg_mean = seg_ref[...] / cnt
    nm = lax.dot_general(oht, seg_mean, (((0,), (0,)), ((), ())),
                         preferred_element_type=jnp.float32)
    diff = static_ref[...] - nm

    @pl.when(pl.program_id(0) == 0)
    def _():
        var_ref[...] = jnp.zeros_like(var_ref)
    var_ref[...] += _dot(oht, diff * diff)


def _tc_p5(static, seg, cnt, batch3):
    return pl.pallas_call(
        _p5_body,
        grid=(NT,),
        in_specs=[
            pl.BlockSpec((TILE, EMB), lambda i: (i, 0)),
            pl.BlockSpec((B, EMB), lambda i: (0, 0)),
            pl.BlockSpec((B, 128), lambda i: (0, 0)),
            pl.BlockSpec((1, 1, TILE), lambda i: (i, 0, 0)),
        ],
        out_specs=pl.BlockSpec((B, EMB), lambda i: (0, 0)),
        out_shape=jax.ShapeDtypeStruct((B, EMB), jnp.float32),
    )(static, seg, cnt, batch3)


def _p6_body(static_ref, pt_ref, u1_ref, noise_ref, seg_ref, cnt_ref, var_ref,
             batch_ref, pool_ref, kl2_ref, kl1_ref):
    oht = _oht(batch_ref)
    onc = lax.dot_general(oht, jnp.ones((B, 1), jnp.float32),
                          (((0,), (0,)), ((), ())),
                          preferred_element_type=jnp.float32)
    counts = cnt_ref[:, 0:1]
    cnt = jnp.maximum(counts, 1.0)
    seg_mean = seg_ref[...] / cnt
    var = var_ref[...] / jnp.maximum(counts - 1.0, 1.0)
    seg_std = jnp.sqrt(jnp.maximum(var, 0.0))
    dg = functools.partial(lax.dot_general,
                           dimension_numbers=(((0,), (0,)), ((), ())),
                           preferred_element_type=jnp.float32)
    node_mean = dg(oht, seg_mean)
    node_std = dg(oht, seg_std)
    static = static_ref[...]

    bias = 0.0001
    u = u1_ref[:, 0:1]
    eps_ = (bias - (1.0 - bias)) * u + (1.0 - bias)
    gate = jnp.log(eps_) - jnp.log(1.0 - eps_) + pt_ref[:, 0:1]
    lam = jax.nn.sigmoid(gate) * onc
    nm = lam * static + (1.0 - lam) * node_mean
    ns = (1.0 - lam) * node_std
    noisy = nm + noise_ref[...] * ns
    e2 = 1e-07
    kl2t = ((nm - node_mean) / (node_std + e2)) ** 2
    kl1t = jnp.mean((ns / (node_std + e2)) ** 2, axis=1, keepdims=True) * onc

    @pl.when(pl.program_id(0) == 0)
    def _():
        pool_ref[...] = jnp.zeros_like(pool_ref)
        kl2_ref[...] = jnp.zeros_like(kl2_ref)
        kl1_ref[...] = jnp.zeros_like(kl1_ref)
    pool_ref[...] += _dot(oht, noisy)
    kl2_ref[...] += _dot(oht, kl2t)
    kl1_ref[:, 0:1] += _dot(oht, kl1t)


def _tc_p6(static, pt, u1p, noise, seg, cnt, var, batch3):
    return pl.pallas_call(
        _p6_body,
        grid=(NT,),
        in_specs=[
            pl.BlockSpec((TILE, EMB), lambda i: (i, 0)),
            pl.BlockSpec((TILE, 128), lambda i: (i, 0)),
            pl.BlockSpec((TILE, 128), lambda i: (i, 0)),
            pl.BlockSpec((TILE, EMB), lambda i: (i, 0)),
            pl.BlockSpec((B, EMB), lambda i: (0, 0)),
            pl.BlockSpec((B, 128), lambda i: (0, 0)),
            pl.BlockSpec((B, EMB), lambda i: (0, 0)),
            pl.BlockSpec((1, 1, TILE), lambda i: (i, 0, 0)),
        ],
        out_specs=[
            pl.BlockSpec((B, EMB), lambda i: (0, 0)),
            pl.BlockSpec((B, EMB), lambda i: (0, 0)),
            pl.BlockSpec((B, 128), lambda i: (0, 0)),
        ],
        out_shape=[
            jax.ShapeDtypeStruct((B, EMB), jnp.float32),
            jax.ShapeDtypeStruct((B, EMB), jnp.float32),
            jax.ShapeDtypeStruct((B, 128), jnp.float32),
        ],
    )(static, pt, u1p, noise, seg, cnt, var, batch3)


def _p7_body(pool_ref, cnt_ref, kl2_ref, kl1_ref, pw_ref, pat_ref,
             pooled_ref, sc_ref):
    cnt = jnp.maximum(cnt_ref[:, 0:1], 1.0)
    pooled = pool_ref[...] / cnt
    pooled_ref[...] = pooled
    kl_loss = (jnp.sum(kl2_ref[...])
               + EMB * 0.5 * jnp.sum(kl1_ref[:, 0:1])) / (B * EMB)
    pred = _dot(pooled, pw_ref[...])[:, 0:1] + pat_ref[1, 0]
    dmat = pat_ref[0:1, :] - pred
    ppl = jnp.mean(dmat * dmat)
    row = lax.broadcasted_iota(jnp.int32, (8, 128), 0)
    col = lax.broadcasted_iota(jnp.int32, (8, 128), 1)
    o = jnp.where(col == 0, kl_loss, ppl)
    sc_ref[...] = jnp.where(row == 0, o, 0.0)


def _tc_p7(pool, cnt, kl2, kl1, pwm, pat):
    return pl.pallas_call(
        _p7_body,
        out_shape=[
            jax.ShapeDtypeStruct((B, EMB), jnp.float32),
            jax.ShapeDtypeStruct((8, 128), jnp.float32),
        ],
    )(pool, cnt, kl2, kl1, pwm, pat)


def _fold(W, b, g, bt):
    s = g * _BNS
    Wt = (W * s[:, None]).T
    bf = b * s + bt
    b8 = jnp.zeros((8, bf.shape[0]), jnp.float32).at[0].set(bf)
    return Wt, b8


def kernel(patient_repr, x, edge_index, batch, params):
    xp = jnp.pad(x, ((0, NP - N), (0, 0)))
    batch_pad = jnp.pad(batch, (0, NP - N), constant_values=PADB)
    batch3 = batch_pad.reshape(NT, 1, TILE)
    pad_ar = jnp.arange(EP - E, dtype=jnp.int32)
    src = jnp.concatenate([edge_index[0], pad_ar % N]
                          ).reshape(16, NIG, IG, ECHUNK)
    dst = jnp.concatenate([edge_index[1], N + pad_ar % (NP - N)]
                          ).reshape(16, NIG, IG, ECHUNK)

    gin = []
    for lp in params['gin']:
        w1t, b1 = _fold(lp['W1'], lp['b1'], lp['g1'], lp['bt1'])
        w2t, b2 = _fold(lp['W2'], lp['b2'], lp['g2'], lp['bt2'])
        epsv = jnp.broadcast_to((1.0 + lp['eps']).astype(jnp.float32)
                                .reshape(1, 1), (8, 128))
        gin.append((w1t, b1, w2t, b2, epsv))
    vnp = []
    for vp in params['vn']:
        w1t, b1 = _fold(vp['W1'], vp['b1'], vp['g1'], vp['bt1'])
        w2t, b2 = _fold(vp['W2'], vp['b2'], vp['g2'], vp['bt2'])
        vnp.append((w1t, b1, w2t, b2))
    cp = params['comp']
    cw1t, cb1 = _fold(cp['W1'], cp['b1'], cp['g'], cp['bt'])
    cb1x = cb1.at[1, 0].set(cp['b2'][0])
    wp = (jnp.zeros((EMB, 128), jnp.float32)
          .at[:, 0].set(cp['W2'][0])
          .at[:, 1].set(patient_repr[0]))
    pwm = jnp.zeros((EMB, 128), jnp.float32).at[:, 0].set(params['pred']['W'][0])
    pat8 = (jnp.zeros((8, EMB), jnp.float32)
            .at[0].set(patient_repr[0])
            .at[1, 0].set(params['pred']['b'][0]))

    u1 = jax.random.uniform(jax.random.fold_in(jax.random.key(0), 1),
                            (N, 1), jnp.float32)
    u1p = jnp.pad(u1, ((0, NP - N), (0, 127)))
    u2 = jax.random.uniform(jax.random.fold_in(jax.random.key(0), 2),
                            (N, EMB), jnp.float32)
    u2p = jnp.pad(u2, ((0, NP - N), (0, 0)))

    h = xp
    vn = jnp.zeros((B, EMB), jnp.float32)
    for l in range(NUM_LAYER):
        ha, hb, seg = _tc_pre(h, vn, batch3)
        agg = _edge_segsum(ha, hb, src, dst)
        w1t, b1, w2t, b2, epsv = gin[l]
        h = _tc_mlp(ha, hb, agg, w1t, b1, w2t, b2, epsv,
                    final=(l == NUM_LAYER - 1))
        if l < NUM_LAYER - 1:
            vw1t, vb1, vw2t, vb2 = vnp[l]
            vn = _tc_vn(seg, vn, vw1t, vb1, vw2t, vb2)

    pt = _tc_p1(h, cw1t, cb1x, wp)
    stats = _tc_p2(pt)
    static, seg, cnt = _tc_p3(h, pt, stats, batch3)
    var = _tc_p5(static, seg, cnt, batch3)
    pool, kl2, kl1 = _tc_p6(static, pt, u1p, u2p, seg, cnt, var, batch3)
    pooled, sc = _tc_p7(pool, cnt, kl2, kl1, pwm, pat8)

    kl_loss = sc[0, 0].reshape(())
    preserve_rate = stats[0, 2].reshape(())
    patient_pred_loss = sc[0, 1].reshape(())
    return (pooled, kl_loss, preserve_rate, patient_pred_loss)
